# Initial kernel scaffold; baseline (speedup 1.0000x reference)
import jax
import jax.numpy as jnp
from jax.experimental import pallas as pl


def kernel(x, adj, w_l_0, w_r_0, b_0, w_l_1, w_r_1, b_1, w_l_2, w_r_2, b_2):
    raise NotImplementedError("write your pallas kernel here")


# speedup vs baseline: 15.5316x; 15.5316x over previous
import functools

import jax
import jax.numpy as jnp
from jax.experimental import pallas as pl
from jax.experimental.pallas import tpu as pltpu

TM = 1024
TK = 1024


def _layer_kernel(adj_ref, hk_ref, hi_ref, wl_ref, wr_ref, b_ref, o_ref,
                  acc_ref, *, act):
    k = pl.program_id(1)

    @pl.when(k == 0)
    def _():
        acc_ref[...] = jnp.zeros_like(acc_ref)

    acc_ref[...] += jnp.dot(adj_ref[...], hk_ref[...],
                            preferred_element_type=jnp.float32)

    @pl.when(k == pl.num_programs(1) - 1)
    def _():
        agg = acc_ref[...].astype(jnp.bfloat16)
        out = (jnp.dot(agg, wl_ref[...], preferred_element_type=jnp.float32)
               + jnp.dot(hi_ref[...], wr_ref[...],
                         preferred_element_type=jnp.float32)
               + b_ref[...])
        if act == "relu":
            out = jnp.maximum(out, 0.0)
        elif act == "log_softmax":
            m = jnp.max(out, axis=-1, keepdims=True)
            s = out - m
            out = s - jnp.log(jnp.sum(jnp.exp(s), axis=-1, keepdims=True))
        o_ref[...] = out.astype(o_ref.dtype)


def _sage_layer(adj, h, wl, wr, b, *, act, out_dtype):
    n = adj.shape[0]
    f_in = h.shape[1]
    f_out = wl.shape[1]
    grid = (n // TM, n // TK)
    return pl.pallas_call(
        functools.partial(_layer_kernel, act=act),
        out_shape=jax.ShapeDtypeStruct((n, f_out), out_dtype),
        grid_spec=pltpu.PrefetchScalarGridSpec(
            num_scalar_prefetch=0,
            grid=grid,
            in_specs=[
                pl.BlockSpec((TM, TK), lambda i, k: (i, k)),
                pl.BlockSpec((TK, f_in), lambda i, k: (k, 0)),
                pl.BlockSpec((TM, f_in), lambda i, k: (i, 0)),
                pl.BlockSpec((f_in, f_out), lambda i, k: (0, 0)),
                pl.BlockSpec((f_in, f_out), lambda i, k: (0, 0)),
                pl.BlockSpec((1, f_out), lambda i, k: (0, 0)),
            ],
            out_specs=pl.BlockSpec((TM, f_out), lambda i, k: (i, 0)),
            scratch_shapes=[pltpu.VMEM((TM, f_in), jnp.float32)],
        ),
        compiler_params=pltpu.CompilerParams(
            dimension_semantics=("parallel", "arbitrary")),
    )(adj, h, h, wl, wr, b)


def kernel(x, adj, w_l_0, w_r_0, b_0, w_l_1, w_r_1, b_1, w_l_2, w_r_2, b_2):
    adj_b = adj.astype(jnp.bfloat16)
    h = x.astype(jnp.bfloat16)

    h = _sage_layer(adj_b, h,
                    w_l_0.astype(jnp.bfloat16), w_r_0.astype(jnp.bfloat16),
                    b_0.astype(jnp.float32),
                    act="relu", out_dtype=jnp.bfloat16)
    h = _sage_layer(adj_b, h,
                    w_l_1.astype(jnp.bfloat16), w_r_1.astype(jnp.bfloat16),
                    b_1.astype(jnp.float32),
                    act="relu", out_dtype=jnp.bfloat16)
    out = _sage_layer(adj_b, h,
                      w_l_2.astype(jnp.bfloat16), w_r_2.astype(jnp.bfloat16),
                      b_2.astype(jnp.float32),
                      act="log_softmax", out_dtype=jnp.float32)
    return out


# speedup vs baseline: 17.9422x; 1.1552x over previous
import functools

import jax
import jax.numpy as jnp
from jax.experimental import pallas as pl
from jax.experimental.pallas import tpu as pltpu

TM = 1024
TK = 1024


def _epilogue(acc, hi_ref, wl_ref, wr_ref, b_ref, act):
    agg = acc.astype(jnp.bfloat16)
    out = (jnp.dot(agg, wl_ref[...], preferred_element_type=jnp.float32)
           + jnp.dot(hi_ref[...], wr_ref[...],
                     preferred_element_type=jnp.float32)
           + b_ref[...])
    if act == "relu":
        out = jnp.maximum(out, 0.0)
    elif act == "log_softmax":
        m = jnp.max(out, axis=-1, keepdims=True)
        s = out - m
        out = s - jnp.log(jnp.sum(jnp.exp(s), axis=-1, keepdims=True))
    return out


def _layer0_kernel(adj_ref, hk_ref, hi_ref, wl_ref, wr_ref, b_ref,
                   o_ref, adjb_ref, acc_ref, *, act):
    k = pl.program_id(1)

    @pl.when(k == 0)
    def _():
        acc_ref[...] = jnp.zeros_like(acc_ref)

    ab = adj_ref[...].astype(jnp.bfloat16)
    adjb_ref[...] = ab
    acc_ref[...] += jnp.dot(ab, hk_ref[...],
                            preferred_element_type=jnp.float32)

    @pl.when(k == pl.num_programs(1) - 1)
    def _():
        out = _epilogue(acc_ref[...], hi_ref, wl_ref, wr_ref, b_ref, act)
        o_ref[...] = out.astype(o_ref.dtype)


def _layer_kernel(adj_ref, hk_ref, hi_ref, wl_ref, wr_ref, b_ref, o_ref,
                  acc_ref, *, act):
    k = pl.program_id(1)

    @pl.when(k == 0)
    def _():
        acc_ref[...] = jnp.zeros_like(acc_ref)

    acc_ref[...] += jnp.dot(adj_ref[...], hk_ref[...],
                            preferred_element_type=jnp.float32)

    @pl.when(k == pl.num_programs(1) - 1)
    def _():
        out = _epilogue(acc_ref[...], hi_ref, wl_ref, wr_ref, b_ref, act)
        o_ref[...] = out.astype(o_ref.dtype)


def _common_specs(f_in, f_out):
    return [
        pl.BlockSpec((TK, f_in), lambda i, k: (k, 0)),
        pl.BlockSpec((TM, f_in), lambda i, k: (i, 0)),
        pl.BlockSpec((f_in, f_out), lambda i, k: (0, 0)),
        pl.BlockSpec((f_in, f_out), lambda i, k: (0, 0)),
        pl.BlockSpec((1, f_out), lambda i, k: (0, 0)),
    ]


def _sage_layer0(adj_f32, h, wl, wr, b, *, act, out_dtype):
    n = adj_f32.shape[0]
    f_in = h.shape[1]
    f_out = wl.shape[1]
    return pl.pallas_call(
        functools.partial(_layer0_kernel, act=act),
        out_shape=(jax.ShapeDtypeStruct((n, f_out), out_dtype),
                   jax.ShapeDtypeStruct((n, n), jnp.bfloat16)),
        grid_spec=pltpu.PrefetchScalarGridSpec(
            num_scalar_prefetch=0,
            grid=(n // TM, n // TK),
            in_specs=[pl.BlockSpec((TM, TK), lambda i, k: (i, k))]
            + _common_specs(f_in, f_out),
            out_specs=(pl.BlockSpec((TM, f_out), lambda i, k: (i, 0)),
                       pl.BlockSpec((TM, TK), lambda i, k: (i, k))),
            scratch_shapes=[pltpu.VMEM((TM, f_in), jnp.float32)],
        ),
        compiler_params=pltpu.CompilerParams(
            dimension_semantics=("parallel", "arbitrary")),
    )(adj_f32, h, h, wl, wr, b)


def _sage_layer(adj_b, h, wl, wr, b, *, act, out_dtype):
    n = adj_b.shape[0]
    f_in = h.shape[1]
    f_out = wl.shape[1]
    return pl.pallas_call(
        functools.partial(_layer_kernel, act=act),
        out_shape=jax.ShapeDtypeStruct((n, f_out), out_dtype),
        grid_spec=pltpu.PrefetchScalarGridSpec(
            num_scalar_prefetch=0,
            grid=(n // TM, n // TK),
            in_specs=[pl.BlockSpec((TM, TK), lambda i, k: (i, k))]
            + _common_specs(f_in, f_out),
            out_specs=pl.BlockSpec((TM, f_out), lambda i, k: (i, 0)),
            scratch_shapes=[pltpu.VMEM((TM, f_in), jnp.float32)],
        ),
        compiler_params=pltpu.CompilerParams(
            dimension_semantics=("parallel", "arbitrary")),
    )(adj_b, h, h, wl, wr, b)


def kernel(x, adj, w_l_0, w_r_0, b_0, w_l_1, w_r_1, b_1, w_l_2, w_r_2, b_2):
    h = x.astype(jnp.bfloat16)

    h, adj_b = _sage_layer0(
        adj, h,
        w_l_0.astype(jnp.bfloat16), w_r_0.astype(jnp.bfloat16),
        b_0.astype(jnp.float32),
        act="relu", out_dtype=jnp.bfloat16)
    h = _sage_layer(adj_b, h,
                    w_l_1.astype(jnp.bfloat16), w_r_1.astype(jnp.bfloat16),
                    b_1.astype(jnp.float32),
                    act="relu", out_dtype=jnp.bfloat16)
    out = _sage_layer(adj_b, h,
                      w_l_2.astype(jnp.bfloat16), w_r_2.astype(jnp.bfloat16),
                      b_2.astype(jnp.float32),
                      act="log_softmax", out_dtype=jnp.float32)
    return out


# speedup vs baseline: 20.1606x; 1.1236x over previous
import functools

import jax
import jax.numpy as jnp
from jax.experimental import pallas as pl
from jax.experimental.pallas import tpu as pltpu

TM = 1024
TK = 2048


def _epilogue(acc, hi_ref, wl_ref, wr_ref, b_ref, act):
    agg = acc.astype(jnp.bfloat16)
    out = (jnp.dot(agg, wl_ref[...], preferred_element_type=jnp.float32)
           + jnp.dot(hi_ref[...], wr_ref[...],
                     preferred_element_type=jnp.float32)
           + b_ref[...])
    if act == "relu":
        out = jnp.maximum(out, 0.0)
    elif act == "log_softmax":
        m = jnp.max(out, axis=-1, keepdims=True)
        s = out - m
        out = s - jnp.log(jnp.sum(jnp.exp(s), axis=-1, keepdims=True))
    return out


def _layer0_kernel(adj_ref, hk_ref, hi_ref, wl_ref, wr_ref, b_ref,
                   o_ref, adjb_ref, acc_ref, *, act):
    k = pl.program_id(1)

    @pl.when(k == 0)
    def _():
        acc_ref[...] = jnp.zeros_like(acc_ref)

    ab = adj_ref[...].astype(jnp.bfloat16)
    adjb_ref[...] = ab
    acc_ref[...] += jnp.dot(ab, hk_ref[...],
                            preferred_element_type=jnp.float32)

    @pl.when(k == pl.num_programs(1) - 1)
    def _():
        out = _epilogue(acc_ref[...], hi_ref, wl_ref, wr_ref, b_ref, act)
        o_ref[...] = out.astype(o_ref.dtype)


def _layer_kernel(adj_ref, hk_ref, hi_ref, wl_ref, wr_ref, b_ref, o_ref,
                  acc_ref, *, act):
    k = pl.program_id(1)

    @pl.when(k == 0)
    def _():
        acc_ref[...] = jnp.zeros_like(acc_ref)

    acc_ref[...] += jnp.dot(adj_ref[...], hk_ref[...],
                            preferred_element_type=jnp.float32)

    @pl.when(k == pl.num_programs(1) - 1)
    def _():
        out = _epilogue(acc_ref[...], hi_ref, wl_ref, wr_ref, b_ref, act)
        o_ref[...] = out.astype(o_ref.dtype)


def _common_specs(f_in, f_out):
    return [
        pl.BlockSpec((TK, f_in), lambda i, k: (k, 0)),
        pl.BlockSpec((TM, f_in), lambda i, k: (i, 0)),
        pl.BlockSpec((f_in, f_out), lambda i, k: (0, 0)),
        pl.BlockSpec((f_in, f_out), lambda i, k: (0, 0)),
        pl.BlockSpec((1, f_out), lambda i, k: (0, 0)),
    ]


def _sage_layer0(adj_f32, h, wl, wr, b, *, act, out_dtype):
    n = adj_f32.shape[0]
    f_in = h.shape[1]
    f_out = wl.shape[1]
    return pl.pallas_call(
        functools.partial(_layer0_kernel, act=act),
        out_shape=(jax.ShapeDtypeStruct((n, f_out), out_dtype),
                   jax.ShapeDtypeStruct((n, n), jnp.bfloat16)),
        grid_spec=pltpu.PrefetchScalarGridSpec(
            num_scalar_prefetch=0,
            grid=(n // TM, n // TK),
            in_specs=[pl.BlockSpec((TM, TK), lambda i, k: (i, k))]
            + _common_specs(f_in, f_out),
            out_specs=(pl.BlockSpec((TM, f_out), lambda i, k: (i, 0)),
                       pl.BlockSpec((TM, TK), lambda i, k: (i, k))),
            scratch_shapes=[pltpu.VMEM((TM, f_in), jnp.float32)],
        ),
        compiler_params=pltpu.CompilerParams(
            dimension_semantics=("parallel", "arbitrary")),
    )(adj_f32, h, h, wl, wr, b)


def _sage_layer(adj_b, h, wl, wr, b, *, act, out_dtype):
    n = adj_b.shape[0]
    f_in = h.shape[1]
    f_out = wl.shape[1]
    return pl.pallas_call(
        functools.partial(_layer_kernel, act=act),
        out_shape=jax.ShapeDtypeStruct((n, f_out), out_dtype),
        grid_spec=pltpu.PrefetchScalarGridSpec(
            num_scalar_prefetch=0,
            grid=(n // TM, n // TK),
            in_specs=[pl.BlockSpec((TM, TK), lambda i, k: (i, k))]
            + _common_specs(f_in, f_out),
            out_specs=pl.BlockSpec((TM, f_out), lambda i, k: (i, 0)),
            scratch_shapes=[pltpu.VMEM((TM, f_in), jnp.float32)],
        ),
        compiler_params=pltpu.CompilerParams(
            dimension_semantics=("parallel", "arbitrary")),
    )(adj_b, h, h, wl, wr, b)


def kernel(x, adj, w_l_0, w_r_0, b_0, w_l_1, w_r_1, b_1, w_l_2, w_r_2, b_2):
    h = x.astype(jnp.bfloat16)

    h, adj_b = _sage_layer0(
        adj, h,
        w_l_0.astype(jnp.bfloat16), w_r_0.astype(jnp.bfloat16),
        b_0.astype(jnp.float32),
        act="relu", out_dtype=jnp.bfloat16)
    h = _sage_layer(adj_b, h,
                    w_l_1.astype(jnp.bfloat16), w_r_1.astype(jnp.bfloat16),
                    b_1.astype(jnp.float32),
                    act="relu", out_dtype=jnp.bfloat16)
    out = _sage_layer(adj_b, h,
                      w_l_2.astype(jnp.bfloat16), w_r_2.astype(jnp.bfloat16),
                      b_2.astype(jnp.float32),
                      act="log_softmax", out_dtype=jnp.float32)
    return out


# speedup vs baseline: 20.7858x; 1.0310x over previous
import functools

import jax
import jax.numpy as jnp
from jax.experimental import pallas as pl
from jax.experimental.pallas import tpu as pltpu

TM = 1024
TK = 2048


def _epilogue(acc, hi_ref, wl_ref, wr_ref, b_ref, act):
    agg = acc.astype(jnp.bfloat16)
    out = (jnp.dot(agg, wl_ref[...], preferred_element_type=jnp.float32)
           + jnp.dot(hi_ref[...], wr_ref[...],
                     preferred_element_type=jnp.float32)
           + b_ref[...])
    if act == "relu":
        out = jnp.maximum(out, 0.0)
    elif act == "log_softmax":
        m = jnp.max(out, axis=-1, keepdims=True)
        s = out - m
        out = s - jnp.log(jnp.sum(jnp.exp(s), axis=-1, keepdims=True))
    return out


def _layer0_kernel(adj_ref, hk_ref, hi_ref, wl_ref, wr_ref, b_ref,
                   o_ref, adjb_ref, acc_ref, *, act):
    k = pl.program_id(1)

    @pl.when(k == 0)
    def _():
        acc_ref[...] = jnp.zeros_like(acc_ref)

    ab = adj_ref[...].astype(jnp.bfloat16)
    adjb_ref[...] = ab
    acc_ref[...] += jnp.dot(ab, hk_ref[...],
                            preferred_element_type=jnp.float32)

    @pl.when(k == pl.num_programs(1) - 1)
    def _():
        out = _epilogue(acc_ref[...], hi_ref, wl_ref, wr_ref, b_ref, act)
        o_ref[...] = out.astype(o_ref.dtype)


def _layer_kernel_1k(adj_ref, hk_ref, hi_ref, wl_ref, wr_ref, b_ref, o_ref,
                     *, act):
    acc = jnp.dot(adj_ref[...], hk_ref[...],
                  preferred_element_type=jnp.float32)
    out = _epilogue(acc, hi_ref, wl_ref, wr_ref, b_ref, act)
    o_ref[...] = out.astype(o_ref.dtype)


def _common_specs(f_in, f_out):
    return [
        pl.BlockSpec((TK, f_in), lambda i, k: (k, 0)),
        pl.BlockSpec((TM, f_in), lambda i, k: (i, 0)),
        pl.BlockSpec((f_in, f_out), lambda i, k: (0, 0)),
        pl.BlockSpec((f_in, f_out), lambda i, k: (0, 0)),
        pl.BlockSpec((1, f_out), lambda i, k: (0, 0)),
    ]


def _sage_layer0(adj_f32, h, wl, wr, b, *, act, out_dtype):
    n = adj_f32.shape[0]
    f_in = h.shape[1]
    f_out = wl.shape[1]
    return pl.pallas_call(
        functools.partial(_layer0_kernel, act=act),
        out_shape=(jax.ShapeDtypeStruct((n, f_out), out_dtype),
                   jax.ShapeDtypeStruct((n, n), jnp.bfloat16)),
        grid_spec=pltpu.PrefetchScalarGridSpec(
            num_scalar_prefetch=0,
            grid=(n // TM, n // TK),
            in_specs=[pl.BlockSpec((TM, TK), lambda i, k: (i, k))]
            + _common_specs(f_in, f_out),
            out_specs=(pl.BlockSpec((TM, f_out), lambda i, k: (i, 0)),
                       pl.BlockSpec((TM, TK), lambda i, k: (i, k))),
            scratch_shapes=[pltpu.VMEM((TM, f_in), jnp.float32)],
        ),
        compiler_params=pltpu.CompilerParams(
            dimension_semantics=("parallel", "arbitrary")),
    )(adj_f32, h, h, wl, wr, b)


def _sage_layer(adj_b, h, wl, wr, b, *, act, out_dtype):
    n = adj_b.shape[0]
    f_in = h.shape[1]
    f_out = wl.shape[1]
    return pl.pallas_call(
        functools.partial(_layer_kernel_1k, act=act),
        out_shape=jax.ShapeDtypeStruct((n, f_out), out_dtype),
        grid_spec=pltpu.PrefetchScalarGridSpec(
            num_scalar_prefetch=0,
            grid=(n // TM,),
            in_specs=[
                pl.BlockSpec((TM, n), lambda i: (i, 0)),
                pl.BlockSpec((n, f_in), lambda i: (0, 0)),
                pl.BlockSpec((TM, f_in), lambda i: (i, 0)),
                pl.BlockSpec((f_in, f_out), lambda i: (0, 0)),
                pl.BlockSpec((f_in, f_out), lambda i: (0, 0)),
                pl.BlockSpec((1, f_out), lambda i: (0, 0)),
            ],
            out_specs=pl.BlockSpec((TM, f_out), lambda i: (i, 0)),
        ),
        compiler_params=pltpu.CompilerParams(
            dimension_semantics=("parallel",)),
    )(adj_b, h, h, wl, wr, b)


def kernel(x, adj, w_l_0, w_r_0, b_0, w_l_1, w_r_1, b_1, w_l_2, w_r_2, b_2):
    h = x.astype(jnp.bfloat16)

    h, adj_b = _sage_layer0(
        adj, h,
        w_l_0.astype(jnp.bfloat16), w_r_0.astype(jnp.bfloat16),
        b_0.astype(jnp.float32),
        act="relu", out_dtype=jnp.bfloat16)
    h = _sage_layer(adj_b, h,
                    w_l_1.astype(jnp.bfloat16), w_r_1.astype(jnp.bfloat16),
                    b_1.astype(jnp.float32),
                    act="relu", out_dtype=jnp.bfloat16)
    out = _sage_layer(adj_b, h,
                      w_l_2.astype(jnp.bfloat16), w_r_2.astype(jnp.bfloat16),
                      b_2.astype(jnp.float32),
                      act="log_softmax", out_dtype=jnp.float32)
    return out
